# baseline (speedup 1.0000x reference)
import functools

import jax
import jax.numpy as jnp
from jax import lax
from jax.experimental import pallas as pl
from jax.experimental.pallas import tpu as pltpu
from jax.experimental.pallas import tpu_sc as plsc

_NC = 2
_NS = 16
_K = 80


def _make_segsum(n, e, d):
    nw = _NC * _NS
    epw = e // nw
    nstep = epw // _K
    assert nstep * _K * nw == e
    wchunk = 80
    nchunk = n // wchunk
    assert nchunk * wchunk == n

    mesh = plsc.VectorSubcoreMesh(core_axis_name="c", subcore_axis_name="s")

    assert _K % 8 == 0
    npair = nstep // 2
    tail = nstep - 2 * npair

    def body(p_hbm, src_hbm, dst_hbm, zero_hbm, out_hbm,
             idx_s, idx_d, r0, r1,
             acc, g0, g1, s0, s1):
        rows = (r0, r1)
        semg = (g0, g1)
        sems = (s0, s1)
        cid = lax.axis_index("c")
        sid = lax.axis_index("s")
        wid = sid * _NC + cid

        pltpu.sync_copy(zero_hbm, r0.at[pl.ds(0, wchunk)])

        def zloop(t, carry):
            c = sid + t * _NS

            @pl.when(c < nchunk)
            def _():
                off = pl.multiple_of(c * wchunk, wchunk)
                pltpu.sync_copy(r0.at[pl.ds(0, wchunk)],
                                acc.at[pl.ds(off, wchunk)])
            return carry

        lax.fori_loop(0, (nchunk + _NS - 1) // _NS, zloop, 0)
        plsc.subcore_barrier()

        def sl(j):
            return pl.ds(pl.multiple_of(j * _K, _K), _K)

        pltpu.sync_copy(src_hbm.at[wid], idx_s)
        pltpu.sync_copy(dst_hbm.at[wid], idx_d)
        pltpu.async_copy(p_hbm.at[idx_s.at[sl(0)]], rows[0], semg[0])

        def step(j, u):
            uu = (u + 1) % 2
            pltpu.make_async_copy(p_hbm.at[idx_s.at[sl(j)]],
                                  rows[u], semg[u]).wait()

            @pl.when(j + 1 < nstep)
            def _():
                pltpu.async_copy(p_hbm.at[idx_s.at[sl(j + 1)]],
                                 rows[uu], semg[uu])

            pltpu.sync_copy(rows[u], acc.at[idx_d.at[sl(j)]], add=True)

        def pair(t, carry):
            for u in range(2):
                step(2 * t + u, u)
            return carry

        lax.fori_loop(0, npair, pair, 0)
        for u in range(tail):
            step(nstep - tail + u, u)
        plsc.subcore_barrier()

        def wb(t, carry):
            c = sid + t * _NS

            @pl.when(c < nchunk)
            def _():
                off = pl.multiple_of(c * wchunk, wchunk)
                pltpu.sync_copy(acc.at[pl.ds(off, wchunk)],
                                r0.at[pl.ds(0, wchunk)])
                pltpu.sync_copy(r0.at[pl.ds(0, wchunk)],
                                out_hbm.at[cid, pl.ds(off, wchunk)])
            return carry

        lax.fori_loop(0, (nchunk + _NS - 1) // _NS, wb, 0)

    kern = pl.kernel(
        body,
        out_type=jax.ShapeDtypeStruct((_NC, n, d), jnp.float32),
        mesh=mesh,
        scratch_types=(
            [pltpu.VMEM((epw,), jnp.int32),
             pltpu.VMEM((epw,), jnp.int32)]
            + [pltpu.VMEM((_K, d), jnp.float32)] * 2
            + [pltpu.VMEM_SHARED((n, d), jnp.float32)]
            + [pltpu.SemaphoreType.DMA] * 4
        ),
    )
    return kern


def _mm(x, w):
    return jnp.dot(x.astype(jnp.bfloat16), w.astype(jnp.bfloat16),
                   preferred_element_type=jnp.float32)


def _tc_layer(h, a0, a1, w, b, relu, rblk=1000):
    n, d = h.shape

    def body(h_ref, a0_ref, a1_ref, w_ref, b_ref, o_ref):
        ht = jnp.concatenate([h_ref[...], a0_ref[...] + a1_ref[...]], axis=1)
        o = _mm(ht, w_ref[...]) + b_ref[...]
        o_ref[...] = jnp.maximum(o, 0.0) if relu else o

    return pl.pallas_call(
        body,
        grid=(n // rblk,),
        in_specs=[
            pl.BlockSpec((rblk, d), lambda i: (i, 0)),
            pl.BlockSpec((rblk, d), lambda i: (i, 0)),
            pl.BlockSpec((rblk, d), lambda i: (i, 0)),
            pl.BlockSpec((2 * d, d), lambda i: (0, 0)),
            pl.BlockSpec((1, d), lambda i: (0, 0)),
        ],
        out_specs=pl.BlockSpec((rblk, d), lambda i: (i, 0)),
        out_shape=jax.ShapeDtypeStruct((n, d), jnp.float32),
    )(h, a0, a1, w, b)


def _tc_head(r1, w2, b2, w3, b3, rblk=1000):
    n, d = r1.shape

    def body(r1_ref, w2_ref, b2_ref, w3_ref, b3_ref, o_ref):
        r2 = jnp.maximum(_mm(r1_ref[...], w2_ref[...]) + b2_ref[...], 0.0)
        o_ref[...] = _mm(r2, w3_ref[...]) + b3_ref[...]

    return pl.pallas_call(
        body,
        grid=(n // rblk,),
        in_specs=[
            pl.BlockSpec((rblk, d), lambda i: (i, 0)),
            pl.BlockSpec((d, d), lambda i: (0, 0)),
            pl.BlockSpec((1, d), lambda i: (0, 0)),
            pl.BlockSpec((d, 1), lambda i: (0, 0)),
            pl.BlockSpec((1, 1), lambda i: (0, 0)),
        ],
        out_specs=pl.BlockSpec((rblk, 1), lambda i: (i, 0)),
        out_shape=jax.ShapeDtypeStruct((n, 1), jnp.float32),
    )(r1, w2, b2, w3, b3)


def kernel(node_feat, edge_index, edge_feat,
           W1, b1, Wm1, bm1, Wm2, bm2, Wm3, bm3, Wm4, bm4,
           Wr1, br1, Wr2, br2, Wr3, br3):
    n, d = node_feat.shape
    e = edge_index.shape[1]
    nw = _NC * _NS

    perm = jnp.argsort(edge_index[1], stable=True)
    src_r = edge_index[0][perm].reshape(nw, e // nw)
    dst_r = edge_index[1][perm].reshape(nw, e // nw)
    zeros_hbm = jnp.zeros((80, d), jnp.float32)

    segsum = _make_segsum(n, e, d)

    h = node_feat
    for (w, b) in ((W1, b1), (Wm1, bm1), (Wm2, bm2), (Wm3, bm3), (Wm4, bm4),
                   (Wr1, br1)):
        a = segsum(h, src_r, dst_r, zeros_hbm)
        h = _tc_layer(h, a[0], a[1], w, b.reshape(1, d), relu=True)
    return _tc_head(h, Wr2, br2.reshape(1, d), Wr3, br3.reshape(1, 1))
